# Initial kernel scaffold; baseline (speedup 1.0000x reference)
import functools

import jax
import jax.numpy as jnp
from jax import lax
from jax.experimental import pallas as pl
from jax.experimental.pallas import tpu as pltpu
from jax.experimental.pallas import tpu_sc as plsc

NC = 2
NS = 16
NW = NC * NS
LN_EPS = 1e-5

_MESH = plsc.VectorSubcoreMesh(
    core_axis_name="c", subcore_axis_name="s", num_cores=NC, num_subcores=NS
)


def _sc_degree(edge_index, n):
    e = edge_index.shape[1]
    k = 80
    e_w = e // NW
    steps = e_w // k
    rows = n // NS
    ones = jnp.ones((k, 16), jnp.float32)
    zeros = jnp.zeros((rows, 16), jnp.float32)

    @functools.partial(
        pl.kernel,
        out_type=jax.ShapeDtypeStruct((NC, n, 16), jnp.float32),
        mesh=_MESH,
        scratch_types=[
            pltpu.VMEM_SHARED((n, 16), jnp.float32),
            pltpu.VMEM((rows, 16), jnp.float32),
            pltpu.VMEM((k, 16), jnp.float32),
            pltpu.VMEM((k,), jnp.int32),
        ],
    )
    def deg_kernel(edge_hbm, ones_hbm, zeros_hbm, out_hbm, acc_sh, z_v, ones_v,
                   idx_v):
        c = lax.axis_index("c")
        s = lax.axis_index("s")
        wid = c * NS + s
        pltpu.sync_copy(zeros_hbm, z_v)
        pltpu.sync_copy(ones_hbm, ones_v)
        pltpu.sync_copy(z_v, acc_sh.at[pl.ds(s * rows, rows)])
        plsc.subcore_barrier()

        base = wid * e_w

        @pl.loop(0, steps)
        def _(i):
            pltpu.sync_copy(edge_hbm.at[1, pl.ds(base + i * k, k)], idx_v)
            pltpu.sync_copy(ones_v, acc_sh.at[idx_v], add=True)

        plsc.subcore_barrier()
        pltpu.sync_copy(
            acc_sh.at[pl.ds(s * rows, rows)],
            out_hbm.at[c, pl.ds(s * rows, rows)],
        )

    return deg_kernel(edge_index, ones, zeros)


def _sc_scatter_rows(edge_index, hs):
    e = edge_index.shape[1]
    n, d = hs.shape
    k = 80
    e_w = e // NW
    steps = e_w // k
    rows = n // NS
    zrows = rows // 5
    zeros = jnp.zeros((zrows, d), jnp.float32)

    @functools.partial(
        pl.kernel,
        out_type=jax.ShapeDtypeStruct((NC, n, d), jnp.float32),
        mesh=_MESH,
        scratch_types=[
            pltpu.VMEM_SHARED((n, d), jnp.float32),
            pltpu.VMEM((zrows, d), jnp.float32),
            pltpu.VMEM((k, d), jnp.float32),
            pltpu.VMEM((k,), jnp.int32),
            pltpu.VMEM((k,), jnp.int32),
        ],
    )
    def scat_kernel(edge_hbm, hs_hbm, zeros_hbm, out_hbm, acc_sh, z_v, msg_v,
                    src_v, dst_v):
        c = lax.axis_index("c")
        s = lax.axis_index("s")
        wid = c * NS + s
        pltpu.sync_copy(zeros_hbm, z_v)

        @pl.loop(0, rows // zrows)
        def _(j):
            pltpu.sync_copy(z_v, acc_sh.at[pl.ds(s * rows + j * zrows, zrows)])

        plsc.subcore_barrier()

        base = wid * e_w

        @pl.loop(0, steps)
        def _(i):
            pltpu.sync_copy(edge_hbm.at[0, pl.ds(base + i * k, k)], src_v)
            pltpu.sync_copy(edge_hbm.at[1, pl.ds(base + i * k, k)], dst_v)
            pltpu.sync_copy(hs_hbm.at[src_v], msg_v)
            pltpu.sync_copy(msg_v, acc_sh.at[dst_v], add=True)

        plsc.subcore_barrier()
        pltpu.sync_copy(
            acc_sh.at[pl.ds(s * rows, rows)],
            out_hbm.at[c, pl.ds(s * rows, rows)],
        )

    return scat_kernel(edge_index, hs, zeros)


_BLK = 500


def _row_spec(d):
    return pl.BlockSpec((_BLK, d), lambda i: (i, 0))


def _full_spec(shape):
    nd = len(shape)
    return pl.BlockSpec(shape, lambda i, _n=nd: (0,) * _n)


def _tc1_body(x_ref, degp_ref, w0_ref, l0w_ref, l0b_ref, hs0_ref, res0_ref,
              dinv_ref):
    deg = degp_ref[0][:, 0:1] + degp_ref[1][:, 0:1] + 1.0
    dinv = lax.rsqrt(deg)
    x = x_ref[...]
    h0 = jnp.dot(x, w0_ref[...], preferred_element_type=jnp.float32)
    dinv_b = jnp.broadcast_to(dinv, h0.shape)
    hs0_ref[...] = h0 * dinv_b
    res0_ref[...] = (
        jnp.dot(x, l0w_ref[...], preferred_element_type=jnp.float32)
        + l0b_ref[...]
    )
    dinv_ref[...] = dinv_b


def _ln_relu(t, g, b):
    mu = jnp.mean(t, axis=-1, keepdims=True)
    var = jnp.mean((t - mu) ** 2, axis=-1, keepdims=True)
    return jnp.maximum((t - mu) * lax.rsqrt(var + LN_EPS) * g + b, 0.0)


def _tc2_body(accp_ref, hs0_ref, res0_ref, dinv_ref, b0_ref, g0_ref, be0_ref,
              w1_ref, l1w_ref, l1b_ref, hs1_ref, res1_ref):
    dinv_b = dinv_ref[...]
    gcn0 = (accp_ref[0] + accp_ref[1] + hs0_ref[...]) * dinv_b + b0_ref[...]
    h1 = _ln_relu(gcn0 + res0_ref[...], g0_ref[...], be0_ref[...])
    hs1_ref[...] = (
        jnp.dot(h1, w1_ref[...], preferred_element_type=jnp.float32) * dinv_b
    )
    res1_ref[...] = (
        jnp.dot(h1, l1w_ref[...], preferred_element_type=jnp.float32)
        + l1b_ref[...]
    )


def _tc3_body(accp_ref, hs1_ref, res1_ref, dinv_ref, b1_ref, g1_ref, be1_ref,
              out_ref):
    gcn1 = ((accp_ref[0] + accp_ref[1] + hs1_ref[...]) * dinv_ref[...]
            + b1_ref[...])
    out_ref[...] = _ln_relu(gcn1 + res1_ref[...], g1_ref[...], be1_ref[...])


def kernel(x, edge_index, W0, b0, L0W, L0b, g0, be0, W1, b1, L1W, L1b, g1,
           be1):
    n, d = x.shape
    grid = (n // _BLK,)
    row = _row_spec(d)
    mat = _full_spec((d, d))
    vec = _full_spec((1, d))
    f32 = jnp.float32
    rows_out = jax.ShapeDtypeStruct((n, d), f32)

    degp = _sc_degree(edge_index, n)

    hs0, res0, dinv_b = pl.pallas_call(
        _tc1_body,
        grid=grid,
        in_specs=[
            row,
            pl.BlockSpec((NC, _BLK, 16), lambda i: (0, i, 0)),
            mat, mat, vec,
        ],
        out_specs=[row, row, row],
        out_shape=[rows_out, rows_out, rows_out],
    )(x, degp, W0, L0W, L0b.reshape(1, d))

    accp0 = _sc_scatter_rows(edge_index, hs0)

    hs1, res1 = pl.pallas_call(
        _tc2_body,
        grid=grid,
        in_specs=[
            pl.BlockSpec((NC, _BLK, d), lambda i: (0, i, 0)),
            row, row, row, vec, vec, vec, mat, mat, vec,
        ],
        out_specs=[row, row],
        out_shape=[rows_out, rows_out],
    )(accp0, hs0, res0, dinv_b, b0.reshape(1, d), g0.reshape(1, d),
      be0.reshape(1, d), W1, L1W, L1b.reshape(1, d))

    accp1 = _sc_scatter_rows(edge_index, hs1)

    out = pl.pallas_call(
        _tc3_body,
        grid=grid,
        in_specs=[
            pl.BlockSpec((NC, _BLK, d), lambda i: (0, i, 0)),
            row, row, row, vec, vec, vec,
        ],
        out_specs=row,
        out_shape=rows_out,
    )(accp1, hs1, res1, dinv_b, b1.reshape(1, d), g1.reshape(1, d),
      be1.reshape(1, d))

    return out


# speedup vs baseline: 12.3481x; 12.3481x over previous
import functools

import jax
import jax.numpy as jnp
from jax import lax
from jax.experimental import pallas as pl
from jax.experimental.pallas import tpu as pltpu
from jax.experimental.pallas import tpu_sc as plsc

NC = 2
NS = 16
NW = NC * NS
LN_EPS = 1e-5

def _sc_mesh():
    return plsc.VectorSubcoreMesh(
        core_axis_name="c", subcore_axis_name="s", num_cores=NC,
        num_subcores=NS,
    )


def _pad_rows(n):
    q = 128 * NS
    return ((n + q - 1) // q) * q


def _sc_degree(dst, n):
    e = dst.shape[0]
    k = 80
    e_w = e // NW
    steps = e_w // k
    npad = _pad_rows(n)
    rows = npad // NS
    zrows = 128
    ones = jnp.ones((k, 128), jnp.float32)
    zeros = jnp.zeros((zrows, 128), jnp.float32)

    @functools.partial(
        pl.kernel,
        out_type=jax.ShapeDtypeStruct((NC, npad, 128), jnp.float32),
        mesh=_sc_mesh(),
        scratch_types=[
            pltpu.VMEM_SHARED((npad, 128), jnp.float32),
            pltpu.VMEM((zrows, 128), jnp.float32),
            pltpu.VMEM((k, 128), jnp.float32),
            pltpu.VMEM((k,), jnp.int32),
        ],
    )
    def deg_kernel(dst_hbm, ones_hbm, zeros_hbm, out_hbm, acc_sh, z_v, ones_v,
                   idx_v):
        c = lax.axis_index("c")
        s = lax.axis_index("s")
        wid = c * NS + s
        pltpu.sync_copy(zeros_hbm, z_v)
        pltpu.sync_copy(ones_hbm, ones_v)

        @pl.loop(0, rows // zrows)
        def _(j):
            pltpu.sync_copy(z_v, acc_sh.at[pl.ds(s * rows + j * zrows, zrows)])

        plsc.subcore_barrier()

        base = wid * e_w

        @pl.loop(0, steps)
        def _(i):
            pltpu.sync_copy(dst_hbm.at[pl.ds(base + i * k, k)], idx_v)
            pltpu.sync_copy(ones_v, acc_sh.at[idx_v], add=True)

        plsc.subcore_barrier()
        pltpu.sync_copy(
            acc_sh.at[pl.ds(s * rows, rows)],
            out_hbm.at[c, pl.ds(s * rows, rows)],
        )

    return deg_kernel(dst, ones, zeros)


def _sc_scatter_rows(src, dst, hs):
    e = src.shape[0]
    n, d = hs.shape
    k = 80
    e_w = e // NW
    steps = e_w // k
    npad = _pad_rows(n)
    rows = npad // NS
    zrows = 128
    zeros = jnp.zeros((zrows, d), jnp.float32)

    @functools.partial(
        pl.kernel,
        out_type=jax.ShapeDtypeStruct((NC, npad, d), jnp.float32),
        mesh=_sc_mesh(),
        scratch_types=[
            pltpu.VMEM_SHARED((npad, d), jnp.float32),
            pltpu.VMEM((zrows, d), jnp.float32),
            pltpu.VMEM((k, d), jnp.float32),
            pltpu.VMEM((k,), jnp.int32),
            pltpu.VMEM((k,), jnp.int32),
        ],
    )
    def scat_kernel(src_hbm, dst_hbm, hs_hbm, zeros_hbm, out_hbm, acc_sh, z_v,
                    msg_v, src_v, dst_v):
        c = lax.axis_index("c")
        s = lax.axis_index("s")
        wid = c * NS + s
        pltpu.sync_copy(zeros_hbm, z_v)

        @pl.loop(0, rows // zrows)
        def _(j):
            pltpu.sync_copy(z_v, acc_sh.at[pl.ds(s * rows + j * zrows, zrows)])

        plsc.subcore_barrier()

        base = wid * e_w

        @pl.loop(0, steps)
        def _(i):
            pltpu.sync_copy(src_hbm.at[pl.ds(base + i * k, k)], src_v)
            pltpu.sync_copy(dst_hbm.at[pl.ds(base + i * k, k)], dst_v)
            pltpu.sync_copy(hs_hbm.at[src_v], msg_v)
            pltpu.sync_copy(msg_v, acc_sh.at[dst_v], add=True)

        plsc.subcore_barrier()
        pltpu.sync_copy(
            acc_sh.at[pl.ds(s * rows, rows)],
            out_hbm.at[c, pl.ds(s * rows, rows)],
        )

    return scat_kernel(src, dst, hs, zeros)


_BLK = 1000


def _row_spec(d):
    return pl.BlockSpec((_BLK, d), lambda i: (i, 0))


def _full_spec(shape):
    nd = len(shape)
    return pl.BlockSpec(shape, lambda i, _n=nd: (0,) * _n)


def _tc1_body(x_ref, degp_ref, w0_ref, l0w_ref, l0b_ref, hs0_ref, res0_ref,
              dinv_ref):
    deg = degp_ref[0][:, 0:1] + degp_ref[1][:, 0:1] + 1.0
    dinv = lax.rsqrt(deg)
    x = x_ref[...]
    h0 = jnp.dot(x, w0_ref[...], preferred_element_type=jnp.float32)
    dinv_b = jnp.broadcast_to(dinv, h0.shape)
    hs0_ref[...] = h0 * dinv_b
    res0_ref[...] = (
        jnp.dot(x, l0w_ref[...], preferred_element_type=jnp.float32)
        + l0b_ref[...]
    )
    dinv_ref[...] = dinv_b


def _ln_relu(t, g, b):
    mu = jnp.mean(t, axis=-1, keepdims=True)
    var = jnp.mean((t - mu) ** 2, axis=-1, keepdims=True)
    return jnp.maximum((t - mu) * lax.rsqrt(var + LN_EPS) * g + b, 0.0)


def _tc2_body(accp_ref, hs0_ref, res0_ref, dinv_ref, b0_ref, g0_ref, be0_ref,
              w1_ref, l1w_ref, l1b_ref, hs1_ref, res1_ref):
    dinv_b = dinv_ref[...]
    gcn0 = (accp_ref[0] + accp_ref[1] + hs0_ref[...]) * dinv_b + b0_ref[...]
    h1 = _ln_relu(gcn0 + res0_ref[...], g0_ref[...], be0_ref[...])
    hs1_ref[...] = (
        jnp.dot(h1, w1_ref[...], preferred_element_type=jnp.float32) * dinv_b
    )
    res1_ref[...] = (
        jnp.dot(h1, l1w_ref[...], preferred_element_type=jnp.float32)
        + l1b_ref[...]
    )


def _tc3_body(accp_ref, hs1_ref, res1_ref, dinv_ref, b1_ref, g1_ref, be1_ref,
              out_ref):
    gcn1 = ((accp_ref[0] + accp_ref[1] + hs1_ref[...]) * dinv_ref[...]
            + b1_ref[...])
    out_ref[...] = _ln_relu(gcn1 + res1_ref[...], g1_ref[...], be1_ref[...])


def kernel(x, edge_index, W0, b0, L0W, L0b, g0, be0, W1, b1, L1W, L1b, g1,
           be1):
    n, d = x.shape
    grid = (n // _BLK,)
    row = _row_spec(d)
    mat = _full_spec((d, d))
    vec = _full_spec((1, d))
    f32 = jnp.float32
    rows_out = jax.ShapeDtypeStruct((n, d), f32)

    src = edge_index[0]
    dst = edge_index[1]
    degp = _sc_degree(dst, n)

    hs0, res0, dinv_b = pl.pallas_call(
        _tc1_body,
        grid=grid,
        in_specs=[
            row,
            pl.BlockSpec((NC, _BLK, d), lambda i: (0, i, 0)),
            mat, mat, vec,
        ],
        out_specs=[row, row, row],
        out_shape=[rows_out, rows_out, rows_out],
    )(x, degp, W0, L0W, L0b.reshape(1, d))

    accp0 = _sc_scatter_rows(src, dst, hs0)

    hs1, res1 = pl.pallas_call(
        _tc2_body,
        grid=grid,
        in_specs=[
            pl.BlockSpec((NC, _BLK, d), lambda i: (0, i, 0)),
            row, row, row, vec, vec, vec, mat, mat, vec,
        ],
        out_specs=[row, row],
        out_shape=[rows_out, rows_out],
    )(accp0, hs0, res0, dinv_b, b0.reshape(1, d), g0.reshape(1, d),
      be0.reshape(1, d), W1, L1W, L1b.reshape(1, d))

    accp1 = _sc_scatter_rows(src, dst, hs1)

    out = pl.pallas_call(
        _tc3_body,
        grid=grid,
        in_specs=[
            pl.BlockSpec((NC, _BLK, d), lambda i: (0, i, 0)),
            row, row, row, vec, vec, vec,
        ],
        out_specs=row,
        out_shape=rows_out,
    )(accp1, hs1, res1, dinv_b, b1.reshape(1, d), g1.reshape(1, d),
      be1.reshape(1, d))

    return out
